# baseline (speedup 1.0000x reference)
import functools

import jax
import jax.numpy as jnp
from jax import lax
from jax.experimental import pallas as pl
from jax.experimental.pallas import tpu as pltpu
from jax.experimental.pallas import tpu_sc as plsc

_U = 2048
_I = 16384
_K = 10
_THR = 0.2
_BU = 256
_NB = _U // _BU
_BC = 1024
_NC = _U // _BC
_KC = _I // 16
_BD = 512
_IC = _I // 8


def _tri_ij(t, n):
    i = jnp.zeros((), jnp.int32)
    off = 0
    for m in range(1, n):
        off += n - (m - 1)
        i = i + (t >= off).astype(jnp.int32)
    offs_i = i * n - (i * (i - 1)) // 2
    j = t - offs_i + i
    return i, j


def _rowsum_kernel(tb_ref, out_ref):
    out_ref[...] = jnp.sum(tb_ref[...].astype(jnp.float32), axis=1,
                           keepdims=True)


def _jacnum_kernel(a_ref, bt_ref, up_ref, lo_ref):
    num = jnp.dot(a_ref[...], bt_ref[...],
                  preferred_element_type=jnp.float32)
    up_ref[...] = num
    lo_ref[...] = num.T


def _topkw_kernel(up_ref, lo_ref, rcol_ref, rrow_ref, w_ref):
    i = pl.program_id(0)
    cols = jax.lax.broadcasted_iota(jnp.int32, (_BU, _U), 1)
    num = jnp.where(cols >= i * _BU, up_ref[...], lo_ref[...])
    den = rcol_ref[...] + rrow_ref[...] - num
    den = jnp.where(den == 0.0, 1.0, den)
    jacv = num / den
    rows = i * _BU + jax.lax.broadcasted_iota(jnp.int32, (_BU, _U), 0)
    x = jnp.where(rows == cols, 0.0, jacv)
    mask = (x > _THR).astype(jnp.float32)
    counts = jnp.sum(mask, axis=1, keepdims=True)
    acc = jnp.zeros_like(x)
    y = x
    for _ in range(_K):
        m = jnp.max(y, axis=1, keepdims=True)
        first = jnp.min(jnp.where(y == m, cols, _U), axis=1, keepdims=True)
        onehot = cols == first
        acc = jnp.where(onehot, 1.0, acc)
        y = jnp.where(onehot, -1.0, y)
    w_ref[...] = jnp.where(counts >= float(_K), mask, acc).astype(jnp.bfloat16)


def _d_kernel(w_ref, tb_ref, trow_ref, d_ref, p_ref):
    w = w_ref[...]
    wsum = jnp.sum(w.astype(jnp.float32), axis=1, keepdims=True)
    nm = jnp.dot(w, tb_ref[...], preferred_element_type=jnp.float32)
    nm = nm / jnp.maximum(wsum, 1.0)
    d = 0.5 * trow_ref[...].astype(jnp.float32) + 0.5 * nm
    d_ref[...] = d
    p_ref[...] = jnp.sum(d * d, axis=1, keepdims=True)[None]


def _normfin_kernel(p_ref, n_ref):
    s = jnp.sum(p_ref[...], axis=0)
    n_ref[...] = jnp.maximum(jnp.sqrt(s), 1e-12)


def _cosnum_kernel(a_ref, b_ref, up_ref, lo_ref, acc_ref):
    k = pl.program_id(1)

    @pl.when(k == 0)
    def _init():
        acc_ref[...] = jnp.zeros_like(acc_ref)

    acc_ref[...] += jnp.dot(a_ref[...], b_ref[...],
                            preferred_element_type=jnp.float32)

    @pl.when(k == pl.num_programs(1) - 1)
    def _fin():
        num = acc_ref[...]
        up_ref[...] = num
        lo_ref[...] = num.T


def _sc_topk2_body(cup_hbm, clo_hbm, n_hbm, vals_hbm, idx_hbm,
                   up_v, lo_v, n_v, tmpv, tmpi):
    wid = lax.axis_index("s") * 2 + lax.axis_index("c")
    pltpu.sync_copy(n_hbm, n_v)
    lanes = lax.iota(jnp.int32, 16)
    for g in range(4):
        r0 = wid * 64 + g * 16
        pltpu.sync_copy(cup_hbm.at[pl.ds(r0, 16), :], up_v)
        pltpu.sync_copy(clo_hbm.at[pl.ds(r0, 16), :], lo_v)
        n_i = n_v[pl.ds(r0, 16)]
        cutoff = (r0 // _BC) * _BC

        def make_prescale(slab_ref):
            def body(cc, _):
                nj = n_v[pl.ds(cc * 16, 16)]
                for l in range(16):
                    slab_ref[l, pl.ds(cc * 16, 16)] = (
                        slab_ref[l, pl.ds(cc * 16, 16)] / n_i[l]) / nj
                return 0
            return body

        lax.fori_loop(cutoff // 16, _U // 16, make_prescale(up_v), 0)
        lax.fori_loop(0, cutoff // 16, make_prescale(lo_v), 0)

        def make_body(slab_ref):
            def body(c4, carry):
                for u in range(8):
                    c = c4 * 8 + u
                    s = carry[:_K]
                    si = carry[_K:]
                    x = plsc.load_gather(
                        slab_ref, [lanes, jnp.full((16,), c, jnp.int32)])
                    pred = jnp.any(x > s[_K - 1])

                    def do_insert(op, c=c):
                        xv, s_t, si_t = op
                        s_l = list(s_t)
                        si_l = list(si_t)
                        xi = jnp.full((16,), c, jnp.int32)
                        for t in range(_K):
                            b = xv > s_l[t]
                            s_l[t], xv = (jnp.where(b, xv, s_l[t]),
                                          jnp.where(b, s_l[t], xv))
                            si_l[t], xi = (jnp.where(b, xi, si_l[t]),
                                           jnp.where(b, si_l[t], xi))
                        return tuple(s_l), tuple(si_l)

                    def no_insert(op):
                        _, s_t, si_t = op
                        return s_t, si_t

                    s2, si2 = lax.cond(pred, do_insert, no_insert,
                                       (x, tuple(s), tuple(si)))
                    carry = s2 + si2
                return carry
            return body

        carry = tuple([jnp.full((16,), -2.0, jnp.float32)] * _K
                      + [jnp.zeros((16,), jnp.int32)] * _K)
        carry = lax.fori_loop(0, cutoff // 8, make_body(lo_v), carry)
        carry = lax.fori_loop(cutoff // 8, _U // 8, make_body(up_v), carry)

        for t in range(_K):
            tmpv[t] = carry[t]
            tmpi[t] = carry[_K + t]
        for t in range(_K):
            pltpu.sync_copy(tmpv.at[t], vals_hbm.at[t, pl.ds(r0, 16)])
            pltpu.sync_copy(tmpi.at[t], idx_hbm.at[t, pl.ds(r0, 16)])


_sc_topk2 = pl.kernel(
    _sc_topk2_body,
    out_type=[
        jax.ShapeDtypeStruct((_K, _U), jnp.float32),
        jax.ShapeDtypeStruct((_K, _U), jnp.int32),
    ],
    mesh=plsc.VectorSubcoreMesh(core_axis_name="c", subcore_axis_name="s"),
    compiler_params=pltpu.CompilerParams(needs_layout_passes=False),
    scratch_types=[
        pltpu.VMEM((16, _U), jnp.float32),
        pltpu.VMEM((16, _U), jnp.float32),
        pltpu.VMEM((_U,), jnp.float32),
        pltpu.VMEM((_K, 16), jnp.float32),
        pltpu.VMEM((_K, 16), jnp.int32),
    ],
)


def kernel(train_mat):
    f32 = jnp.float32
    tb = train_mat.astype(jnp.bfloat16)
    tbt = tb.T

    r = pl.pallas_call(
        _rowsum_kernel,
        grid=(_NB,),
        in_specs=[pl.BlockSpec((_BU, _I), lambda i: (i, 0))],
        out_specs=pl.BlockSpec((_BU, 1), lambda i: (i, 0)),
        out_shape=jax.ShapeDtypeStruct((_U, 1), f32),
    )(tb)
    rrow = r.reshape(1, _U)

    ntri8 = _NB * (_NB + 1) // 2

    def _ij8(t):
        return _tri_ij(t, _NB)

    jup, jlo = pl.pallas_call(
        _jacnum_kernel,
        grid=(ntri8,),
        in_specs=[
            pl.BlockSpec((_BU, _I), lambda t: (_ij8(t)[0], 0)),
            pl.BlockSpec((_I, _BU), lambda t: (0, _ij8(t)[1])),
        ],
        out_specs=[
            pl.BlockSpec((_BU, _BU), lambda t: _ij8(t)),
            pl.BlockSpec((_BU, _BU), lambda t: (_ij8(t)[1], _ij8(t)[0])),
        ],
        out_shape=[
            jax.ShapeDtypeStruct((_U, _U), f32),
            jax.ShapeDtypeStruct((_U, _U), f32),
        ],
    )(tb, tbt)

    w = pl.pallas_call(
        _topkw_kernel,
        grid=(_NB,),
        in_specs=[
            pl.BlockSpec((_BU, _U), lambda i: (i, 0)),
            pl.BlockSpec((_BU, _U), lambda i: (i, 0)),
            pl.BlockSpec((_BU, 1), lambda i: (i, 0)),
            pl.BlockSpec((1, _U), lambda i: (0, 0)),
        ],
        out_specs=pl.BlockSpec((_BU, _U), lambda i: (i, 0)),
        out_shape=jax.ShapeDtypeStruct((_U, _U), jnp.bfloat16),
    )(jup, jlo, r, rrow)

    d, p = pl.pallas_call(
        _d_kernel,
        grid=(8, _U // _BD),
        in_specs=[
            pl.BlockSpec((_BD, _U), lambda j, i: (i, 0)),
            pl.BlockSpec((_U, _IC), lambda j, i: (0, j)),
            pl.BlockSpec((_BD, _IC), lambda j, i: (i, j)),
        ],
        out_specs=[
            pl.BlockSpec((_BD, _IC), lambda j, i: (i, j)),
            pl.BlockSpec((1, _BD, 1), lambda j, i: (j, i, 0)),
        ],
        out_shape=[
            jax.ShapeDtypeStruct((_U, _I), f32),
            jax.ShapeDtypeStruct((8, _U, 1), f32),
        ],
    )(w, tb, tb)

    n = pl.pallas_call(
        _normfin_kernel,
        grid=(1,),
        in_specs=[pl.BlockSpec((8, _U, 1), lambda i: (0, 0, 0))],
        out_specs=pl.BlockSpec((_U, 1), lambda i: (0, 0)),
        out_shape=jax.ShapeDtypeStruct((_U, 1), f32),
    )(p)

    dt = d.T
    ntri4 = _NC * (_NC + 1) // 2

    def _ij4(t):
        return _tri_ij(t, _NC)

    cup, clo = pl.pallas_call(
        _cosnum_kernel,
        grid=(ntri4, _I // _KC),
        in_specs=[
            pl.BlockSpec((_BC, _KC), lambda t, k: (_ij4(t)[0], k)),
            pl.BlockSpec((_KC, _BC), lambda t, k: (k, _ij4(t)[1])),
        ],
        out_specs=[
            pl.BlockSpec((_BC, _BC), lambda t, k: _ij4(t)),
            pl.BlockSpec((_BC, _BC), lambda t, k: (_ij4(t)[1], _ij4(t)[0])),
        ],
        out_shape=[
            jax.ShapeDtypeStruct((_U, _U), f32),
            jax.ShapeDtypeStruct((_U, _U), f32),
        ],
        scratch_shapes=[pltpu.VMEM((_BC, _BC), f32)],
    )(d, dt)

    vals_t, idx_t = _sc_topk2(cup, clo, n.reshape(_U))
    return vals_t.T, idx_t.T


# speedup vs baseline: 1.0356x; 1.0356x over previous
import functools

import jax
import jax.numpy as jnp
from jax import lax
from jax.experimental import pallas as pl
from jax.experimental.pallas import tpu as pltpu
from jax.experimental.pallas import tpu_sc as plsc

_U = 2048
_I = 16384
_K = 10
_THR = 0.2
_BU = 256
_NB = _U // _BU
_BC = 1024
_NC = _U // _BC
_KC = _I // 16
_BD = 512
_IC = _I // 8


def _tri_ij(t, n):
    i = jnp.zeros((), jnp.int32)
    off = 0
    for m in range(1, n):
        off += n - (m - 1)
        i = i + (t >= off).astype(jnp.int32)
    offs_i = i * n - (i * (i - 1)) // 2
    j = t - offs_i + i
    return i, j


def _rowsum_kernel(tb_ref, out_ref):
    out_ref[...] = jnp.sum(tb_ref[...].astype(jnp.float32), axis=1,
                           keepdims=True)


def _jacnum_kernel(a_ref, bt_ref, up_ref, lo_ref):
    num = jnp.dot(a_ref[...], bt_ref[...],
                  preferred_element_type=jnp.float32)
    up_ref[...] = num
    lo_ref[...] = num.T


def _topkw_kernel(up_ref, lo_ref, rcol_ref, rrow_ref, w_ref):
    i = pl.program_id(0)
    cols = jax.lax.broadcasted_iota(jnp.int32, (_BU, _U), 1)
    num = jnp.where(cols >= i * _BU, up_ref[...], lo_ref[...])
    den = rcol_ref[...] + rrow_ref[...] - num
    den = jnp.where(den == 0.0, 1.0, den)
    jacv = num / den
    rows = i * _BU + jax.lax.broadcasted_iota(jnp.int32, (_BU, _U), 0)
    x = jnp.where(rows == cols, 0.0, jacv)
    mask = (x > _THR).astype(jnp.float32)
    counts = jnp.sum(mask, axis=1, keepdims=True)
    acc = jnp.zeros_like(x)
    y = x
    for _ in range(_K):
        m = jnp.max(y, axis=1, keepdims=True)
        first = jnp.min(jnp.where(y == m, cols, _U), axis=1, keepdims=True)
        onehot = cols == first
        acc = jnp.where(onehot, 1.0, acc)
        y = jnp.where(onehot, -1.0, y)
    w_ref[...] = jnp.where(counts >= float(_K), mask, acc).astype(jnp.bfloat16)


def _d_kernel(w_ref, tb_ref, trow_ref, d_ref, p_ref):
    w = w_ref[...]
    wsum = jnp.sum(w.astype(jnp.float32), axis=1, keepdims=True)
    nm = jnp.dot(w, tb_ref[...], preferred_element_type=jnp.float32)
    nm = nm / jnp.maximum(wsum, 1.0)
    d = 0.5 * trow_ref[...].astype(jnp.float32) + 0.5 * nm
    d_ref[...] = d
    p_ref[...] = jnp.sum(d * d, axis=1, keepdims=True)[None]


def _normfin_kernel(p_ref, n_ref):
    s = jnp.sum(p_ref[...], axis=0)
    n_ref[...] = jnp.maximum(jnp.sqrt(s), 1e-12)


def _cosnum_kernel(a_ref, b_ref, up_ref, lo_ref, acc_ref):
    k = pl.program_id(1)

    @pl.when(k == 0)
    def _init():
        acc_ref[...] = jnp.zeros_like(acc_ref)

    acc_ref[...] += jnp.dot(a_ref[...], b_ref[...],
                            preferred_element_type=jnp.float32)

    @pl.when(k == pl.num_programs(1) - 1)
    def _fin():
        num = acc_ref[...]
        up_ref[...] = num
        lo_ref[...] = num.T


def _sc_topk2_body(cup_hbm, clo_hbm, n_hbm, vals_hbm, idx_hbm,
                   up_v, lo_v, n_v, tmpv, tmpi):
    wid = lax.axis_index("s") * 2 + lax.axis_index("c")
    pltpu.sync_copy(n_hbm, n_v)
    lanes = lax.iota(jnp.int32, 16)
    for g in range(4):
        r0 = wid * 64 + g * 16
        pltpu.sync_copy(cup_hbm.at[pl.ds(r0, 16), :], up_v)
        pltpu.sync_copy(clo_hbm.at[pl.ds(r0, 16), :], lo_v)
        n_i = n_v[pl.ds(r0, 16)]
        cutoff = (r0 // _BC) * _BC

        def make_prescale(slab_ref):
            def body(cc, _):
                nj = n_v[pl.ds(cc * 16, 16)]
                for l in range(16):
                    slab_ref[l, pl.ds(cc * 16, 16)] = (
                        slab_ref[l, pl.ds(cc * 16, 16)] / n_i[l]) / nj
                return 0
            return body

        lax.fori_loop(cutoff // 16, _U // 16, make_prescale(up_v), 0)
        lax.fori_loop(0, cutoff // 16, make_prescale(lo_v), 0)

        def make_body(slab_ref):
            def body(c4, carry):
                for u in range(4):
                    c = c4 * 4 + u
                    s = carry[:_K]
                    si = carry[_K:]
                    x = plsc.load_gather(
                        slab_ref, [lanes, jnp.full((16,), c, jnp.int32)])
                    pred = jnp.any(x > s[_K - 1])

                    def do_insert(op, c=c):
                        xv, s_t, si_t = op
                        s_l = list(s_t)
                        si_l = list(si_t)
                        xi = jnp.full((16,), c, jnp.int32)
                        for t in range(_K):
                            b = xv > s_l[t]
                            s_l[t], xv = (jnp.where(b, xv, s_l[t]),
                                          jnp.where(b, s_l[t], xv))
                            si_l[t], xi = (jnp.where(b, xi, si_l[t]),
                                           jnp.where(b, si_l[t], xi))
                        return tuple(s_l), tuple(si_l)

                    def no_insert(op):
                        _, s_t, si_t = op
                        return s_t, si_t

                    s2, si2 = lax.cond(pred, do_insert, no_insert,
                                       (x, tuple(s), tuple(si)))
                    carry = s2 + si2
                return carry
            return body

        carry = tuple([jnp.full((16,), -2.0, jnp.float32)] * _K
                      + [jnp.zeros((16,), jnp.int32)] * _K)
        carry = lax.fori_loop(0, cutoff // 4, make_body(lo_v), carry)
        carry = lax.fori_loop(cutoff // 4, _U // 4, make_body(up_v), carry)

        for t in range(_K):
            tmpv[t] = carry[t]
            tmpi[t] = carry[_K + t]
        for t in range(_K):
            pltpu.sync_copy(tmpv.at[t], vals_hbm.at[t, pl.ds(r0, 16)])
            pltpu.sync_copy(tmpi.at[t], idx_hbm.at[t, pl.ds(r0, 16)])


_sc_topk2 = pl.kernel(
    _sc_topk2_body,
    out_type=[
        jax.ShapeDtypeStruct((_K, _U), jnp.float32),
        jax.ShapeDtypeStruct((_K, _U), jnp.int32),
    ],
    mesh=plsc.VectorSubcoreMesh(core_axis_name="c", subcore_axis_name="s"),
    compiler_params=pltpu.CompilerParams(needs_layout_passes=False),
    scratch_types=[
        pltpu.VMEM((16, _U), jnp.float32),
        pltpu.VMEM((16, _U), jnp.float32),
        pltpu.VMEM((_U,), jnp.float32),
        pltpu.VMEM((_K, 16), jnp.float32),
        pltpu.VMEM((_K, 16), jnp.int32),
    ],
)


def kernel(train_mat):
    f32 = jnp.float32
    tb = train_mat.astype(jnp.bfloat16)
    tbt = tb.T

    r = pl.pallas_call(
        _rowsum_kernel,
        grid=(_NB,),
        in_specs=[pl.BlockSpec((_BU, _I), lambda i: (i, 0))],
        out_specs=pl.BlockSpec((_BU, 1), lambda i: (i, 0)),
        out_shape=jax.ShapeDtypeStruct((_U, 1), f32),
    )(tb)
    rrow = r.reshape(1, _U)

    ntri8 = _NB * (_NB + 1) // 2

    def _ij8(t):
        return _tri_ij(t, _NB)

    jup, jlo = pl.pallas_call(
        _jacnum_kernel,
        grid=(ntri8,),
        in_specs=[
            pl.BlockSpec((_BU, _I), lambda t: (_ij8(t)[0], 0)),
            pl.BlockSpec((_I, _BU), lambda t: (0, _ij8(t)[1])),
        ],
        out_specs=[
            pl.BlockSpec((_BU, _BU), lambda t: _ij8(t)),
            pl.BlockSpec((_BU, _BU), lambda t: (_ij8(t)[1], _ij8(t)[0])),
        ],
        out_shape=[
            jax.ShapeDtypeStruct((_U, _U), f32),
            jax.ShapeDtypeStruct((_U, _U), f32),
        ],
    )(tb, tbt)

    w = pl.pallas_call(
        _topkw_kernel,
        grid=(_NB,),
        in_specs=[
            pl.BlockSpec((_BU, _U), lambda i: (i, 0)),
            pl.BlockSpec((_BU, _U), lambda i: (i, 0)),
            pl.BlockSpec((_BU, 1), lambda i: (i, 0)),
            pl.BlockSpec((1, _U), lambda i: (0, 0)),
        ],
        out_specs=pl.BlockSpec((_BU, _U), lambda i: (i, 0)),
        out_shape=jax.ShapeDtypeStruct((_U, _U), jnp.bfloat16),
    )(jup, jlo, r, rrow)

    d, p = pl.pallas_call(
        _d_kernel,
        grid=(8, _U // _BD),
        in_specs=[
            pl.BlockSpec((_BD, _U), lambda j, i: (i, 0)),
            pl.BlockSpec((_U, _IC), lambda j, i: (0, j)),
            pl.BlockSpec((_BD, _IC), lambda j, i: (i, j)),
        ],
        out_specs=[
            pl.BlockSpec((_BD, _IC), lambda j, i: (i, j)),
            pl.BlockSpec((1, _BD, 1), lambda j, i: (j, i, 0)),
        ],
        out_shape=[
            jax.ShapeDtypeStruct((_U, _I), f32),
            jax.ShapeDtypeStruct((8, _U, 1), f32),
        ],
    )(w, tb, tb)

    n = pl.pallas_call(
        _normfin_kernel,
        grid=(1,),
        in_specs=[pl.BlockSpec((8, _U, 1), lambda i: (0, 0, 0))],
        out_specs=pl.BlockSpec((_U, 1), lambda i: (0, 0)),
        out_shape=jax.ShapeDtypeStruct((_U, 1), f32),
    )(p)

    dt = d.T
    ntri4 = _NC * (_NC + 1) // 2

    def _ij4(t):
        return _tri_ij(t, _NC)

    cup, clo = pl.pallas_call(
        _cosnum_kernel,
        grid=(ntri4, _I // _KC),
        in_specs=[
            pl.BlockSpec((_BC, _KC), lambda t, k: (_ij4(t)[0], k)),
            pl.BlockSpec((_KC, _BC), lambda t, k: (k, _ij4(t)[1])),
        ],
        out_specs=[
            pl.BlockSpec((_BC, _BC), lambda t, k: _ij4(t)),
            pl.BlockSpec((_BC, _BC), lambda t, k: (_ij4(t)[1], _ij4(t)[0])),
        ],
        out_shape=[
            jax.ShapeDtypeStruct((_U, _U), f32),
            jax.ShapeDtypeStruct((_U, _U), f32),
        ],
        scratch_shapes=[pltpu.VMEM((_BC, _BC), f32)],
    )(d, dt)

    vals_t, idx_t = _sc_topk2(cup, clo, n.reshape(_U))
    return vals_t.T, idx_t.T
